# Initial kernel scaffold; baseline (speedup 1.0000x reference)
import functools

import jax
import jax.numpy as jnp
from jax import lax
from jax.experimental import pallas as pl
from jax.experimental.pallas import tpu as pltpu
from jax.experimental.pallas import tpu_sc as plsc

B, S, EMB = 4, 2048, 128
NW = 32
RPW = (B * S) // NW
CH = EMB // 16


def _body(word_hbm, idx_hbm, pos_hbm, type_hbm, gamma_hbm, beta_hbm,
          out_hbm, idx_v, rows_v, pos_v, type_v, gamma_v, beta_v, sem):
    wid = lax.axis_index("s") * 2 + lax.axis_index("c")
    base = wid * RPW
    sbase = (wid % (S // RPW)) * RPW

    pltpu.sync_copy(idx_hbm.at[pl.ds(wid * 2, 2)], idx_v)
    cp0 = pltpu.async_copy(word_hbm.at[idx_v.at[0]],
                           rows_v.at[pl.ds(0, 128)], sem)
    cp1 = pltpu.async_copy(word_hbm.at[idx_v.at[1]],
                           rows_v.at[pl.ds(128, 128)], sem)
    pltpu.sync_copy(pos_hbm.at[pl.ds(sbase + 2, RPW)], pos_v)
    pltpu.sync_copy(type_hbm.at[pl.ds(0, 1)], type_v)
    pltpu.sync_copy(gamma_hbm, gamma_v)
    pltpu.sync_copy(beta_hbm, beta_v)
    cp0.wait()
    cp1.wait()

    tch = [type_v[0, pl.ds(16 * c, 16)] for c in range(CH)]
    gch = [gamma_v[pl.ds(16 * c, 16)] for c in range(CH)]
    bch = [beta_v[pl.ds(16 * c, 16)] for c in range(CH)]
    inv_n = jnp.float32(1.0 / EMB)

    def row(r, carry):
        x = [rows_v[r, pl.ds(16 * c, 16)] + pos_v[r, pl.ds(16 * c, 16)]
             + tch[c] for c in range(CH)]
        s0 = (x[0] + x[1]) + (x[2] + x[3])
        s1 = (x[4] + x[5]) + (x[6] + x[7])
        sq0 = (x[0] * x[0] + x[1] * x[1]) + (x[2] * x[2] + x[3] * x[3])
        sq1 = (x[4] * x[4] + x[5] * x[5]) + (x[6] * x[6] + x[7] * x[7])
        tot = jnp.sum(s0 + s1)
        totsq = jnp.sum(sq0 + sq1)
        mean = tot * inv_n
        var = totsq * inv_n - mean * mean
        v = var + jnp.float32(1e-5)
        i = lax.bitcast_convert_type(v, jnp.int32)
        i = jnp.int32(0x5F3759DF) - (i >> 1)
        y = lax.bitcast_convert_type(i, jnp.float32)
        half_v = v * jnp.float32(0.5)
        y = y * (jnp.float32(1.5) - half_v * y * y)
        y = y * (jnp.float32(1.5) - half_v * y * y)
        y = y * (jnp.float32(1.5) - half_v * y * y)
        for c in range(CH):
            a = gch[c] * y
            b = bch[c] - a * mean
            rows_v[r, pl.ds(16 * c, 16)] = x[c] * a + b
        return carry

    lax.fori_loop(0, RPW, row, jnp.int32(0))

    pltpu.sync_copy(rows_v, out_hbm.at[pl.ds(base, RPW)])


@jax.jit
def _run(word_emb, idx2d, pos_emb, type_emb, ln_gamma, ln_beta):
    mesh = plsc.VectorSubcoreMesh(core_axis_name="c", subcore_axis_name="s")
    k = functools.partial(
        pl.kernel,
        mesh=mesh,
        out_type=jax.ShapeDtypeStruct((B * S, EMB), jnp.float32),
        scratch_types=[
            pltpu.VMEM((2, 128), jnp.int32),
            pltpu.VMEM((RPW, EMB), jnp.float32),
            pltpu.VMEM((RPW, EMB), jnp.float32),
            pltpu.VMEM((1, EMB), jnp.float32),
            pltpu.VMEM((EMB,), jnp.float32),
            pltpu.VMEM((EMB,), jnp.float32),
            pltpu.SemaphoreType.DMA,
        ],
    )(_body)
    return k(word_emb, idx2d, pos_emb, type_emb, ln_gamma, ln_beta)


def kernel(input_ids, word_emb, pos_emb, type_emb, ln_gamma, ln_beta):
    idx2d = input_ids.astype(jnp.int32).reshape(NW * 2, 128)
    out = _run(word_emb, idx2d, pos_emb, type_emb, ln_gamma, ln_beta)
    return out.reshape(B, S, EMB)


# speedup vs baseline: 1.0566x; 1.0566x over previous
import functools

import jax
import jax.numpy as jnp
from jax import lax
from jax.experimental import pallas as pl
from jax.experimental.pallas import tpu as pltpu
from jax.experimental.pallas import tpu_sc as plsc

B, S, EMB = 4, 2048, 128
NW = 32
RPW = (B * S) // NW
CH = EMB // 16
NG = RPW // 16


def _body(word_hbm, idx_hbm, pos_hbm, type_hbm, gamma_hbm, beta_hbm,
          out_hbm, idx_v, rows_v, pos_v, type_v, gamma_v, beta_v,
          ssum_v, ssq_v, sem):
    wid = lax.axis_index("s") * 2 + lax.axis_index("c")
    base = wid * RPW
    sbase = (wid % (S // RPW)) * RPW

    pltpu.sync_copy(idx_hbm.at[pl.ds(base, RPW)], idx_v)
    cp0 = pltpu.async_copy(word_hbm.at[idx_v.at[pl.ds(0, 128)]],
                           rows_v.at[pl.ds(0, 128)], sem)
    cp1 = pltpu.async_copy(word_hbm.at[idx_v.at[pl.ds(128, 128)]],
                           rows_v.at[pl.ds(128, 128)], sem)
    pltpu.sync_copy(pos_hbm.at[pl.ds(sbase, RPW)], pos_v)
    pltpu.sync_copy(type_hbm, type_v)
    pltpu.sync_copy(gamma_hbm, gamma_v)
    pltpu.sync_copy(beta_hbm, beta_v)
    cp0.wait()
    cp1.wait()

    inv_n = jnp.float32(1.0 / EMB)
    lane = lax.iota(jnp.int32, 16)
    colbase = lane * 16

    def group(g, carry):
        r0 = g * 16
        for rr in range(16):
            r = r0 + rr
            s = None
            sq = None
            for c in range(CH):
                d = pl.ds(16 * c, 16)
                x = rows_v[r, d] + pos_v[r, d] + type_v[d]
                rows_v[r, d] = x
                xx = x * x
                s = x if s is None else s + x
                sq = xx if sq is None else sq + xx
            ssum_v[pl.ds(16 * rr, 16)] = s
            ssq_v[pl.ds(16 * rr, 16)] = sq
        tot = None
        totsq = None
        for j in range(16):
            idxj = colbase + j
            ts = plsc.load_gather(ssum_v, [idxj])
            tq = plsc.load_gather(ssq_v, [idxj])
            tot = ts if tot is None else tot + ts
            totsq = tq if totsq is None else totsq + tq
        mean = tot * inv_n
        var = totsq * inv_n - mean * mean
        v = var + jnp.float32(1e-5)
        i = plsc.bitcast(v, jnp.int32)
        i = jnp.int32(0x5F3759DF) - (i >> 1)
        y = plsc.bitcast(i, jnp.float32)
        half_v = v * jnp.float32(0.5)
        y = y * (jnp.float32(1.5) - half_v * y * y)
        y = y * (jnp.float32(1.5) - half_v * y * y)
        y = y * (jnp.float32(1.5) - half_v * y * y)
        for rr in range(16):
            r = r0 + rr
            sc = y[rr]
            ms = mean[rr] * sc
            for c in range(CH):
                d = pl.ds(16 * c, 16)
                t = rows_v[r, d] * sc - ms
                rows_v[r, d] = t * gamma_v[d] + beta_v[d]
        return carry

    lax.fori_loop(0, NG, group, jnp.int32(0))

    pltpu.sync_copy(rows_v, out_hbm.at[pl.ds(base, RPW)])


@jax.jit
def _run(word_emb, idx, pos_sl, type_row, ln_gamma, ln_beta):
    mesh = plsc.VectorSubcoreMesh(core_axis_name="c", subcore_axis_name="s")
    k = functools.partial(
        pl.kernel,
        mesh=mesh,
        compiler_params=pltpu.CompilerParams(needs_layout_passes=False),
        out_type=jax.ShapeDtypeStruct((B * S, EMB), jnp.float32),
        scratch_types=[
            pltpu.VMEM((RPW,), jnp.int32),
            pltpu.VMEM((RPW, EMB), jnp.float32),
            pltpu.VMEM((RPW, EMB), jnp.float32),
            pltpu.VMEM((EMB,), jnp.float32),
            pltpu.VMEM((EMB,), jnp.float32),
            pltpu.VMEM((EMB,), jnp.float32),
            pltpu.VMEM((256,), jnp.float32),
            pltpu.VMEM((256,), jnp.float32),
            pltpu.SemaphoreType.DMA,
        ],
    )(_body)
    return k(word_emb, idx, pos_sl, type_row, ln_gamma, ln_beta)


def kernel(input_ids, word_emb, pos_emb, type_emb, ln_gamma, ln_beta):
    idx = input_ids.astype(jnp.int32).reshape(B * S)
    pos_sl = pos_emb[2:2 + S]
    type_row = type_emb[0]
    out = _run(word_emb, idx, pos_sl, type_row, ln_gamma, ln_beta)
    return out.reshape(B, S, EMB)
